# Initial kernel scaffold; baseline (speedup 1.0000x reference)
import jax
import jax.numpy as jnp
from jax.experimental import pallas as pl


def kernel(edge_index, edge_attr, num_nodes, same_frame_edge_index, same_frame_edge_attr, W, b):
    raise NotImplementedError("write your pallas kernel here")


# speedup vs baseline: 1.3675x; 1.3675x over previous
import jax
import jax.numpy as jnp
from jax import lax
from jax.experimental import pallas as pl
from jax.experimental.pallas import tpu as pltpu
from jax.experimental.pallas import tpu_sc as plsc

N_NODES = 100000
E_MAIN = 3200000
E_FRAME = 1600000
D_E = 16
D_OUT = 128

NC = 2
NS = 16

N_PAD = 100352
RPT = N_PAD // NS
ACC_W = RPT * D_E

WIN_MAIN = 1024
WIN_FR = 512
NW_MAIN = E_MAIN // WIN_MAIN
NW_FR = E_FRAME // WIN_FR
RING = 1280
CHUNK = 128


def _sc_body(past_h, fut_h, early_h, later_h, attr_h, sf_h, out_o,
             idxwin, attrwin, rowbuf, locbuf, acc, sem):
    cid = lax.axis_index("c")
    sid = lax.axis_index("s")
    nbase = sid * RPT

    iota = lax.iota(jnp.int32, 16)
    ones = jnp.ones((16,), jnp.int32)
    zerosf = jnp.zeros((16,), jnp.float32)

    def zero_acc():
        def zb(i, c):
            acc[pl.ds(i * 16, 16)] = zerosf
            return c
        lax.fori_loop(0, ACC_W // 16, zb, 0)

    def drain_chunk(c):
        for g in range(8):
            locv = locbuf[pl.ds(c * CHUNK + g * 16, 16)]
            rwv = rowbuf[pl.ds(c * CHUNK + g * 16, 16)]
            m = locv < RPT
            lbase = locv * D_E
            rwbase = rwv * D_E
            for f in range(16):
                vals = plsc.load_gather(attrwin, [rwbase + f])
                plsc.addupdate_scatter(acc, [lbase + f], vals, mask=m)

    def agg_pass(idx_ref, attr_ref, n_win, win):
        nvecs = win // 16

        def wbody(w, c):
            cpa = pltpu.async_copy(
                attr_ref.at[pl.ds(w * win * D_E, win * D_E)],
                attrwin.at[pl.ds(0, win * D_E)], sem)
            pltpu.sync_copy(idx_ref.at[pl.ds(w * win, win)],
                            idxwin.at[pl.ds(0, win)])
            wptr = jnp.zeros((16,), jnp.int32)
            for k in range(nvecs):
                v = idxwin[pl.ds(k * 16, 16)]
                lo = v - nbase
                m = (v >= nbase) & (v < nbase + RPT)
                rank = plsc.cumsum(ones, mask=m)
                addr = wptr + rank - 1
                rowid = k * 16 + iota
                plsc.store_scatter(rowbuf, [addr], rowid, mask=m)
                plsc.store_scatter(locbuf, [addr], lo, mask=m)
                cnt = plsc.all_reduce_population_count(m)
                wptr = wptr + cnt
            wcount = wptr[0]
            for g in range(8):
                rowbuf[pl.ds(wcount + g * 16, 16)] = jnp.zeros((16,),
                                                               jnp.int32)
                locbuf[pl.ds(wcount + g * 16, 16)] = jnp.full((16,), RPT,
                                                              jnp.int32)
            cpa.wait()
            nch = (wcount + CHUNK - 1) // CHUNK

            def dbody(ci, cc):
                drain_chunk(ci)
                return cc
            lax.fori_loop(0, nch, dbody, 0)
            return c

        lax.fori_loop(0, n_win, wbody, 0)

    def copy_out(t):
        pltpu.sync_copy(acc,
                        out_o.at[pl.ds(t * N_PAD * D_E + sid * ACC_W, ACC_W)])

    zero_acc()

    @pl.when(cid == 0)
    def _():
        agg_pass(fut_h, attr_h, NW_MAIN, WIN_MAIN)
        copy_out(0)

    @pl.when(cid == 1)
    def _():
        agg_pass(past_h, attr_h, NW_MAIN, WIN_MAIN)
        copy_out(3)

    zero_acc()

    @pl.when(cid == 0)
    def _():
        agg_pass(early_h, sf_h, NW_FR, WIN_FR)
        copy_out(1)

    @pl.when(cid == 1)
    def _():
        agg_pass(later_h, sf_h, NW_FR, WIN_FR)
        copy_out(2)


def _sc_aggregate(past, fut, early, later, attr, sf):
    mesh = plsc.VectorSubcoreMesh(core_axis_name="c", subcore_axis_name="s")
    scratch = [
        pltpu.VMEM((WIN_MAIN,), jnp.int32),
        pltpu.VMEM((WIN_MAIN * D_E,), jnp.float32),
        pltpu.VMEM((RING,), jnp.int32),
        pltpu.VMEM((RING,), jnp.int32),
        pltpu.VMEM((ACC_W,), jnp.float32),
        pltpu.SemaphoreType.DMA,
    ]
    return pl.kernel(
        _sc_body,
        out_type=jax.ShapeDtypeStruct((4 * N_PAD * D_E,), jnp.float32),
        mesh=mesh,
        scratch_types=scratch,
        compiler_params=pltpu.CompilerParams(needs_layout_passes=False),
    )(past, fut, early, later, attr, sf)


BLK = 2000


def _mlp_body(aggs, w_ref, b_ref, out):
    a = aggs[...]
    x = jnp.concatenate([a[0], a[1] + a[2], a[3]], axis=1)
    out[...] = lax.dot_general(x, w_ref[...], (((1,), (0,)), ((), ())),
                               preferred_element_type=jnp.float32) + b_ref[...]


def _mlp(aggs, W, b):
    grid = (N_NODES // BLK,)
    in_specs = [
        pl.BlockSpec((4, BLK, D_E), lambda i: (0, i, 0)),
        pl.BlockSpec((3 * D_E, D_OUT), lambda i: (0, 0)),
        pl.BlockSpec((1, D_OUT), lambda i: (0, 0)),
    ]
    out_spec = pl.BlockSpec((BLK, D_OUT), lambda i: (i, 0))
    return pl.pallas_call(
        _mlp_body,
        grid=grid,
        in_specs=in_specs,
        out_specs=out_spec,
        out_shape=jax.ShapeDtypeStruct((N_NODES, D_OUT), jnp.float32),
    )(aggs, W, b.reshape(1, D_OUT))


def kernel(edge_index, edge_attr, num_nodes, same_frame_edge_index,
           same_frame_edge_attr, W, b):
    past = edge_index[0]
    future = edge_index[1]
    early = same_frame_edge_index[0]
    later = same_frame_edge_index[1]

    flat = _sc_aggregate(past, future, early, later,
                         edge_attr.reshape(-1),
                         same_frame_edge_attr.reshape(-1))
    aggs = flat.reshape(4, N_PAD, D_E)
    return _mlp(aggs, W, b)
